# baseline (device time: 20180 ns/iter reference)
import jax
import jax.numpy as jnp
from jax import lax
from jax.experimental import pallas as pl
from jax.experimental.pallas import tpu as pltpu

P = 512
H = 256


def kernel(x):
    m, n = x.shape

    def body(x_ref, out_ref, s1_send, s1_recv, s2_send, s2_recv):
        my_x = lax.axis_index("x")
        my_y = lax.axis_index("y")
        ox = 1 - my_x
        oy = 1 - my_y

        barrier_sem = pltpu.get_barrier_semaphore()
        pl.semaphore_signal(
            barrier_sem, inc=1,
            device_id=(ox, my_y), device_id_type=pl.DeviceIdType.MESH,
        )
        pl.semaphore_signal(
            barrier_sem, inc=1,
            device_id=(my_x, oy), device_id_type=pl.DeviceIdType.MESH,
        )
        pl.semaphore_wait(barrier_sem, 2)

        rdma1 = pltpu.make_async_remote_copy(
            src_ref=x_ref.at[pl.ds(my_y * H, H), pl.ds(ox * P, P)],
            dst_ref=out_ref.at[pl.ds(my_x * P + my_y * H, H), :],
            send_sem=s1_send,
            recv_sem=s1_recv,
            device_id=(ox, my_y),
            device_id_type=pl.DeviceIdType.MESH,
        )
        rdma1.start()

        out_ref[pl.ds(my_x * P, P), :] = x_ref[:, pl.ds(my_x * P, P)]

        rdma1.wait_recv()

        rdma2 = pltpu.make_async_remote_copy(
            src_ref=out_ref.at[pl.ds(ox * P + my_y * H, H), :],
            dst_ref=out_ref.at[pl.ds(ox * P + my_y * H, H), :],
            send_sem=s2_send,
            recv_sem=s2_recv,
            device_id=(my_x, oy),
            device_id_type=pl.DeviceIdType.MESH,
        )
        rdma2.start()
        rdma2.wait()
        rdma1.wait_send()

    return pl.pallas_call(
        body,
        out_shape=jax.ShapeDtypeStruct((2 * m, n // 2), x.dtype),
        in_specs=[pl.BlockSpec(memory_space=pltpu.VMEM)],
        out_specs=pl.BlockSpec(memory_space=pltpu.VMEM),
        scratch_shapes=[
            pltpu.SemaphoreType.DMA,
            pltpu.SemaphoreType.DMA,
            pltpu.SemaphoreType.DMA,
            pltpu.SemaphoreType.DMA,
        ],
        compiler_params=pltpu.CompilerParams(collective_id=0),
    )(x)


# device time: 14866 ns/iter; 1.3575x vs baseline; 1.3575x over previous
import jax
import jax.numpy as jnp
from jax import lax
from jax.experimental import pallas as pl
from jax.experimental.pallas import tpu as pltpu

P = 512
F = 192
C = 3
CH = F // C
D = P - 2 * F


def kernel(x):
    m, n = x.shape

    def body(x_ref, out_ref, s1_send, s1_recv, sd_send, sd_recv,
             s2_send, s2_recv):
        my_x = lax.axis_index("x")
        my_y = lax.axis_index("y")
        ox = 1 - my_x
        oy = 1 - my_y

        barrier_sem = pltpu.get_barrier_semaphore()
        pl.semaphore_signal(
            barrier_sem, inc=1,
            device_id=(ox, my_y), device_id_type=pl.DeviceIdType.MESH,
        )
        pl.semaphore_signal(
            barrier_sem, inc=1,
            device_id=(my_x, oy), device_id_type=pl.DeviceIdType.MESH,
        )
        pl.semaphore_wait(barrier_sem, 2)

        fwd_off = (1 - my_y) * (P - F)
        xs = []
        for j in range(C):
            r = pltpu.make_async_remote_copy(
                src_ref=x_ref.at[pl.ds(fwd_off + j * CH, CH),
                                 pl.ds(ox * P, P)],
                dst_ref=out_ref.at[pl.ds(my_x * P + fwd_off + j * CH, CH), :],
                send_sem=s1_send.at[j], recv_sem=s1_recv.at[j],
                device_id=(ox, my_y), device_id_type=pl.DeviceIdType.MESH,
            )
            r.start()
            xs.append(r)

        rd = pltpu.make_async_remote_copy(
            src_ref=x_ref.at[pl.ds(F, D), pl.ds(ox * P, P)],
            dst_ref=out_ref.at[pl.ds(my_x * P + F, D), :],
            send_sem=sd_send, recv_sem=sd_recv,
            device_id=(ox, my_y), device_id_type=pl.DeviceIdType.MESH,
        )
        rd.start()

        out_ref[pl.ds(my_x * P, P), :] = x_ref[:, pl.ds(my_x * P, P)]

        ys = []
        for j in range(C):
            xs[j].wait_recv()
            r = pltpu.make_async_remote_copy(
                src_ref=out_ref.at[pl.ds(ox * P + fwd_off + j * CH, CH), :],
                dst_ref=out_ref.at[pl.ds(ox * P + fwd_off + j * CH, CH), :],
                send_sem=s2_send.at[j], recv_sem=s2_recv.at[j],
                device_id=(my_x, oy), device_id_type=pl.DeviceIdType.MESH,
            )
            r.start()
            ys.append(r)

        rd.wait()
        for j in range(C):
            ys[j].wait()
            xs[j].wait_send()

    return pl.pallas_call(
        body,
        out_shape=jax.ShapeDtypeStruct((2 * m, n // 2), x.dtype),
        in_specs=[pl.BlockSpec(memory_space=pltpu.VMEM)],
        out_specs=pl.BlockSpec(memory_space=pltpu.VMEM),
        scratch_shapes=[
            pltpu.SemaphoreType.DMA((C,)),
            pltpu.SemaphoreType.DMA((C,)),
            pltpu.SemaphoreType.DMA,
            pltpu.SemaphoreType.DMA,
            pltpu.SemaphoreType.DMA((C,)),
            pltpu.SemaphoreType.DMA((C,)),
        ],
        compiler_params=pltpu.CompilerParams(collective_id=0),
    )(x)
